# baseline (device time: 6896 ns/iter reference)
import jax
import jax.numpy as jnp
from jax import lax
from jax.experimental import pallas as pl
from jax.experimental.pallas import tpu as pltpu

X_SIZE = 2


def kernel(x):
    m, n = x.shape
    blk = n // X_SIZE

    def body(
        x_hbm, out_hbm,
        xs_vmem, xl_vmem, send_buf, local_buf,
        in_sem_s, in_sem_l, out_sem, send_sem, recv_sem,
    ):
        my_x = lax.axis_index("x")
        my_y = lax.axis_index("y")
        my_z = lax.axis_index("z")
        peer = (1 - my_x, my_y, my_z)

        barrier_sem = pltpu.get_barrier_semaphore()
        pl.semaphore_signal(
            barrier_sem, inc=1, device_id=peer,
            device_id_type=pl.DeviceIdType.MESH,
        )
        cp_s = pltpu.make_async_copy(
            x_hbm.at[:, pl.ds((1 - my_x) * blk, blk)], xs_vmem, in_sem_s
        )
        cp_s.start()
        cp_l = pltpu.make_async_copy(
            x_hbm.at[:, pl.ds(my_x * blk, blk)], xl_vmem, in_sem_l
        )
        cp_l.start()
        pl.semaphore_wait(barrier_sem, 1)

        cp_s.wait()
        send_buf[:, :] = xs_vmem[:, :].astype(jnp.bfloat16)
        rdma = pltpu.make_async_remote_copy(
            src_ref=send_buf,
            dst_ref=out_hbm.at[pl.ds(my_x * m, m), :],
            send_sem=send_sem,
            recv_sem=recv_sem,
            device_id=peer,
            device_id_type=pl.DeviceIdType.MESH,
        )
        rdma.start()

        cp_l.wait()
        local_buf[:, :] = xl_vmem[:, :].astype(jnp.bfloat16)
        cp_out = pltpu.make_async_copy(
            local_buf, out_hbm.at[pl.ds(my_x * m, m), :], out_sem
        )
        cp_out.start()
        cp_out.wait()

        rdma.wait()

    return pl.pallas_call(
        body,
        out_shape=jax.ShapeDtypeStruct((X_SIZE * m, blk), jnp.bfloat16),
        in_specs=[pl.BlockSpec(memory_space=pl.ANY)],
        out_specs=pl.BlockSpec(memory_space=pl.ANY),
        scratch_shapes=[
            pltpu.VMEM((m, blk), x.dtype),
            pltpu.VMEM((m, blk), x.dtype),
            pltpu.VMEM((m, blk), jnp.bfloat16),
            pltpu.VMEM((m, blk), jnp.bfloat16),
            pltpu.SemaphoreType.DMA,
            pltpu.SemaphoreType.DMA,
            pltpu.SemaphoreType.DMA,
            pltpu.SemaphoreType.DMA,
            pltpu.SemaphoreType.DMA,
        ],
        compiler_params=pltpu.CompilerParams(collective_id=0),
    )(x)


# device time: 6876 ns/iter; 1.0029x vs baseline; 1.0029x over previous
import jax
import jax.numpy as jnp
from jax import lax
from jax.experimental import pallas as pl
from jax.experimental.pallas import tpu as pltpu

X_SIZE = 2


def kernel(x):
    m, n = x.shape
    blk = n // X_SIZE

    def body(
        x_ref, out_ref,
        xs_vmem, xl_vmem, send_buf,
        in_sem_s, in_sem_l, send_sem, recv_sem,
    ):
        my_x = lax.axis_index("x")
        my_y = lax.axis_index("y")
        my_z = lax.axis_index("z")
        peer = (1 - my_x, my_y, my_z)

        barrier_sem = pltpu.get_barrier_semaphore()
        pl.semaphore_signal(
            barrier_sem, inc=1, device_id=peer,
            device_id_type=pl.DeviceIdType.MESH,
        )
        cp_s = pltpu.make_async_copy(
            x_ref.at[:, pl.ds((1 - my_x) * blk, blk)], xs_vmem, in_sem_s
        )
        cp_s.start()
        cp_l = pltpu.make_async_copy(
            x_ref.at[:, pl.ds(my_x * blk, blk)], xl_vmem, in_sem_l
        )
        cp_l.start()
        pl.semaphore_wait(barrier_sem, 1)

        cp_s.wait()
        send_buf[:, :] = xs_vmem[:, :].astype(jnp.bfloat16)
        rdma = pltpu.make_async_remote_copy(
            src_ref=send_buf,
            dst_ref=out_ref.at[pl.ds(my_x * m, m), :],
            send_sem=send_sem,
            recv_sem=recv_sem,
            device_id=peer,
            device_id_type=pl.DeviceIdType.MESH,
        )
        rdma.start()

        cp_l.wait()
        out_ref[pl.ds(my_x * m, m), :] = xl_vmem[:, :].astype(jnp.bfloat16)

        rdma.wait()

    return pl.pallas_call(
        body,
        out_shape=jax.ShapeDtypeStruct((X_SIZE * m, blk), jnp.bfloat16),
        in_specs=[pl.BlockSpec(memory_space=pltpu.VMEM)],
        out_specs=pl.BlockSpec(memory_space=pltpu.VMEM),
        scratch_shapes=[
            pltpu.VMEM((m, blk), x.dtype),
            pltpu.VMEM((m, blk), x.dtype),
            pltpu.VMEM((m, blk), jnp.bfloat16),
            pltpu.SemaphoreType.DMA,
            pltpu.SemaphoreType.DMA,
            pltpu.SemaphoreType.DMA,
            pltpu.SemaphoreType.DMA,
        ],
        compiler_params=pltpu.CompilerParams(collective_id=0),
    )(x)


# device time: 6845 ns/iter; 1.0075x vs baseline; 1.0045x over previous
import jax
import jax.numpy as jnp
from jax import lax
from jax.experimental import pallas as pl
from jax.experimental.pallas import tpu as pltpu

X_SIZE = 2


def kernel(x):
    m, n = x.shape
    blk = n // X_SIZE

    def body(x_ref, out_ref, xb, send_sem, recv_sem):
        my_x = lax.axis_index("x")
        my_y = lax.axis_index("y")
        my_z = lax.axis_index("z")
        peer = (1 - my_x, my_y, my_z)

        barrier_sem = pltpu.get_barrier_semaphore()
        pl.semaphore_signal(
            barrier_sem, inc=1, device_id=peer,
            device_id_type=pl.DeviceIdType.MESH,
        )
        xb[:, :] = x_ref[:, :].astype(jnp.bfloat16)
        pl.semaphore_wait(barrier_sem, 1)

        def exchange(mx):
            rdma = pltpu.make_async_remote_copy(
                src_ref=xb.at[:, (1 - mx) * blk:(2 - mx) * blk],
                dst_ref=out_ref.at[mx * m:(mx + 1) * m, :],
                send_sem=send_sem,
                recv_sem=recv_sem,
                device_id=peer,
                device_id_type=pl.DeviceIdType.MESH,
            )
            rdma.start()
            out_ref[mx * m:(mx + 1) * m, :] = xb[:, mx * blk:(mx + 1) * blk]
            rdma.wait()

        @pl.when(my_x == 0)
        def _():
            exchange(0)

        @pl.when(my_x == 1)
        def _():
            exchange(1)

    return pl.pallas_call(
        body,
        out_shape=jax.ShapeDtypeStruct((X_SIZE * m, blk), jnp.bfloat16),
        in_specs=[pl.BlockSpec(memory_space=pltpu.VMEM)],
        out_specs=pl.BlockSpec(memory_space=pltpu.VMEM),
        scratch_shapes=[
            pltpu.VMEM((m, n), jnp.bfloat16),
            pltpu.SemaphoreType.DMA,
            pltpu.SemaphoreType.DMA,
        ],
        compiler_params=pltpu.CompilerParams(collective_id=0),
    )(x)
